# Initial kernel scaffold; baseline (speedup 1.0000x reference)
import jax
import jax.numpy as jnp
from jax.experimental import pallas as pl


def kernel(inputs, Wr, br, W1, b1, W2, b2):
    raise NotImplementedError("write your pallas kernel here")


# speedup vs baseline: 2.7549x; 2.7549x over previous
import functools

import jax
import jax.numpy as jnp
from jax.experimental import pallas as pl

B, S, D = 2, 2048, 1024
E, FF, K = 8, 2048, 2
BS = B * S
CAPACITY = 640.0
TB = 1024
NT = BS // TB


def _router_kernel(x_ref, wr_ref, br_ref, m_ref, a_ref):
    t = pl.program_id(0)
    x = x_ref[...]
    wr = wr_ref[...]
    logits = jax.lax.dot_general(
        x, wr, (((1,), (1,)), ((), ())), preferred_element_type=jnp.float32
    )
    logits = logits + br_ref[...]
    mx = jnp.max(logits, axis=1, keepdims=True)
    ex = jnp.exp(logits - mx)
    probs = ex / jnp.sum(ex, axis=1, keepdims=True)
    iota = jax.lax.broadcasted_iota(jnp.int32, (TB, E), 1)
    p1 = jnp.max(probs, axis=1, keepdims=True)
    i1 = jnp.argmax(probs, axis=1).reshape(TB, 1)
    masked = jnp.where(iota == i1, -jnp.inf, probs)
    p2 = jnp.max(masked, axis=1, keepdims=True)
    i2 = jnp.argmax(masked, axis=1).reshape(TB, 1)
    s = p1 + p2
    w1 = p1 / s
    w2 = p2 / s
    iota128 = jax.lax.broadcasted_iota(jnp.int32, (TB, 128), 1)
    m = w1 * (iota128 == i1).astype(jnp.float32) + w2 * (iota128 == i2).astype(
        jnp.float32
    )
    m_ref[...] = m
    a1 = jnp.sum(w1 * (iota128 == i1).astype(jnp.float32), axis=0, keepdims=True)
    a2 = jnp.sum(w2 * (iota128 == i2).astype(jnp.float32), axis=0, keepdims=True)
    iota_s = jax.lax.broadcasted_iota(jnp.int32, (8, 128), 0)
    a_part = jnp.where(iota_s == 0, a1, 0.0) + jnp.where(iota_s == 1, a2, 0.0)

    @pl.when(t == 0)
    def _():
        a_ref[...] = a_part

    @pl.when(t != 0)
    def _():
        a_ref[...] += a_part


def _ffn_kernel(x_ref, m_ref, w1_ref, b1_ref, w2_ref, b2_ref, o_ref):
    e = pl.program_id(1)
    x = x_ref[...]
    mcol = jnp.sum(
        m_ref[...] * (jax.lax.broadcasted_iota(jnp.int32, (TB, 128), 1) == e),
        axis=1,
        keepdims=True,
    )
    xs = (x * mcol).astype(jnp.bfloat16)
    w1 = w1_ref[0]
    h = jax.lax.dot_general(
        xs, w1, (((1,), (1,)), ((), ())), preferred_element_type=jnp.float32
    )
    h = h + b1_ref[0]
    h = 0.5 * h * (1.0 + jax.lax.erf(h * 0.7071067811865476))
    hb = h.astype(jnp.bfloat16)
    w2 = w2_ref[0]
    out = jax.lax.dot_general(
        hb, w2, (((1,), (1,)), ((), ())), preferred_element_type=jnp.float32
    )
    out = out + b2_ref[0]
    contrib = mcol * out

    @pl.when(e == 0)
    def _():
        o_ref[...] = contrib

    @pl.when(e != 0)
    def _():
        o_ref[...] += contrib


@functools.partial(jax.jit)
def kernel(inputs, Wr, br, W1, b1, W2, b2):
    b, s, d = inputs.shape
    xf = inputs.reshape(BS, D)
    m, a = pl.pallas_call(
        _router_kernel,
        grid=(NT,),
        in_specs=[
            pl.BlockSpec((TB, D), lambda t: (t, 0)),
            pl.BlockSpec((E, D), lambda t: (0, 0)),
            pl.BlockSpec((1, E), lambda t: (0, 0)),
        ],
        out_specs=[
            pl.BlockSpec((TB, 128), lambda t: (t, 0)),
            pl.BlockSpec((8, 128), lambda t: (0, 0)),
        ],
        out_shape=[
            jax.ShapeDtypeStruct((BS, 128), jnp.float32),
            jax.ShapeDtypeStruct((8, 128), jnp.float32),
        ],
    )(xf, Wr, br.reshape(1, E))
    m = m.at[0:E, 0:K].add(a[0:K, 0:E].T)
    m = jnp.minimum(m, CAPACITY)

    w1b = W1.astype(jnp.bfloat16)
    w2b = W2.astype(jnp.bfloat16)
    out = pl.pallas_call(
        _ffn_kernel,
        grid=(NT, E),
        in_specs=[
            pl.BlockSpec((TB, D), lambda t, e: (t, 0)),
            pl.BlockSpec((TB, 128), lambda t, e: (t, 0)),
            pl.BlockSpec((1, FF, D), lambda t, e: (e, 0, 0)),
            pl.BlockSpec((1, 1, FF), lambda t, e: (e, 0, 0)),
            pl.BlockSpec((1, D, FF), lambda t, e: (e, 0, 0)),
            pl.BlockSpec((1, 1, D), lambda t, e: (e, 0, 0)),
        ],
        out_specs=pl.BlockSpec((TB, D), lambda t, e: (t, 0)),
        out_shape=jax.ShapeDtypeStruct((BS, D), jnp.float32),
    )(xf, m, w1b, b1.reshape(E, 1, FF), w2b, b2.reshape(E, 1, D))
    return out.reshape(b, s, d)
